# Initial kernel scaffold; baseline (speedup 1.0000x reference)
import jax
import jax.numpy as jnp
from jax.experimental import pallas as pl


def kernel(x, sdf, edge_index, edge_indexA2, edge_attr, edge_attrA2, coarse_nodes, coarse_y, params):
    raise NotImplementedError("write your pallas kernel here")


# speedup vs baseline: 2.9107x; 2.9107x over previous
import functools

import jax
import jax.numpy as jnp
from jax import lax
from jax.experimental import pallas as pl
from jax.experimental.pallas import tpu as pltpu
from jax.experimental.pallas import tpu_sc as plsc

N_NODES = 10000
N_EDGES = 160000
N_COARSE = 2000
N_PAD = 10240
E_PAD = 163840
NC_PAD = 2048
DUMMY = 10200

NCORES = 2
NSUB = 16
NW = NCORES * NSUB
CHUNK = 128
QT = E_PAD // NW
NCHUNK = QT // CHUNK
ROWS_T = N_PAD // NSUB


def _mesh():
    return plsc.VectorSubcoreMesh(core_axis_name="c", subcore_axis_name="s",
                                  num_cores=NCORES, num_subcores=NSUB)


_SC_PARAMS = pltpu.CompilerParams(use_tc_tiling_on_sc=False)


def _make_gather(cp):
    @functools.partial(
        pl.kernel,
        out_type=jax.ShapeDtypeStruct((E_PAD, cp), jnp.float32),
        mesh=_mesh(),
        scratch_types=[
            pltpu.VMEM((CHUNK,), jnp.int32),
            pltpu.VMEM((CHUNK, cp), jnp.float32),
            pltpu.SemaphoreType.DMA,
        ],
        compiler_params=_SC_PARAMS,
        name=f"sc_gather_{cp}",
    )
    def gk(table, src, out, idx_v, buf, sem):
        wid = lax.axis_index("s") * NCORES + lax.axis_index("c")
        base = wid * QT

        def body(k, carry):
            off = base + k * CHUNK
            pltpu.sync_copy(src.at[pl.ds(off, CHUNK)], idx_v)
            pltpu.async_copy(table.at[idx_v], buf, sem).wait()
            pltpu.sync_copy(buf, out.at[pl.ds(off, CHUNK)])
            return carry

        lax.fori_loop(0, NCHUNK, body, 0)

    return gk


def _make_scatter(nsets, cp):
    out_types = [jax.ShapeDtypeStruct((NCORES, N_PAD, cp), jnp.float32)
                 for _ in range(nsets)]
    scratch = ([pltpu.VMEM_SHARED((N_PAD, cp), jnp.float32)
                for _ in range(nsets)]
               + [pltpu.VMEM((CHUNK,), jnp.int32),
                  pltpu.VMEM((CHUNK, cp), jnp.float32)])

    @functools.partial(pl.kernel, out_type=out_types, mesh=_mesh(),
                       scratch_types=scratch, compiler_params=_SC_PARAMS,
                       name=f"sc_scatter_{nsets}x{cp}")
    def sk(*refs):
        ins = refs[:2 * nsets]
        outs = refs[2 * nsets:3 * nsets]
        accs = refs[3 * nsets:4 * nsets]
        idx_v = refs[4 * nsets]
        buf = refs[4 * nsets + 1]

        cid = lax.axis_index("c")
        sid = lax.axis_index("s")
        wid = sid * NCORES + cid
        base = wid * QT
        zrow = sid * ROWS_T

        zeros16 = jnp.zeros((16,), jnp.float32)

        def zb(i, carry):
            r = i // (cp // 16)
            j = i % (cp // 16)
            buf[r, pl.ds(j * 16, 16)] = zeros16
            return carry

        lax.fori_loop(0, CHUNK * (cp // 16), zb, 0)
        for a in accs:
            def zs(t, carry, a=a):
                pltpu.sync_copy(buf, a.at[pl.ds(zrow + t * CHUNK, CHUNK)])
                return carry
            lax.fori_loop(0, ROWS_T // CHUNK, zs, 0)
        plsc.subcore_barrier()

        def body(k, carry):
            off = base + k * CHUNK
            for i in range(nsets):
                contrib, dst = ins[2 * i], ins[2 * i + 1]
                pltpu.sync_copy(dst.at[pl.ds(off, CHUNK)], idx_v)
                pltpu.sync_copy(contrib.at[pl.ds(off, CHUNK)], buf)
                pltpu.sync_copy(buf, accs[i].at[idx_v], add=True)
            return carry

        lax.fori_loop(0, NCHUNK, body, 0)
        plsc.subcore_barrier()

        for i in range(nsets):
            pltpu.sync_copy(accs[i].at[pl.ds(zrow, ROWS_T)],
                            outs[i].at[cid, pl.ds(zrow, ROWS_T)])

    return sk


def _make_dense(cin_p, cout_p, be=2048):
    grid = E_PAD // be

    def body(ea_ref, xj_ref, win_ref, bin_ref, wout_ref, out_ref):
        ea = ea_ref[...]
        xj = xj_ref[...]
        acc = jnp.zeros((be, cout_p), jnp.float32)
        for h in range(3):
            s = jnp.maximum(
                jnp.dot(ea, win_ref[h], preferred_element_type=jnp.float32)
                + bin_ref[h], 0.0)
            acc = acc + jnp.dot(s * xj, wout_ref[h],
                                preferred_element_type=jnp.float32)
        out_ref[...] = acc

    return pl.pallas_call(
        body,
        grid=(grid,),
        in_specs=[
            pl.BlockSpec((be, 8), lambda i: (i, 0)),
            pl.BlockSpec((be, cin_p), lambda i: (i, 0)),
            pl.BlockSpec((3, 8, cin_p), lambda i: (0, 0, 0)),
            pl.BlockSpec((3, 1, cin_p), lambda i: (0, 0, 0)),
            pl.BlockSpec((3, cin_p, cout_p), lambda i: (0, 0, 0)),
        ],
        out_specs=pl.BlockSpec((be, cout_p), lambda i: (i, 0)),
        out_shape=jax.ShapeDtypeStruct((E_PAD, cout_p), jnp.float32),
        name=f"tc_dense_{cin_p}_{cout_p}",
    )


BN = 512


def _knn_body(fp_ref, cpos_ref, cy_ref, out_ref):
    f0 = fp_ref[:, 0:1]
    f1 = fp_ref[:, 1:2]
    c0 = cpos_ref[0:1, :]
    c1 = cpos_ref[1:2, :]
    d2 = (f0 - c0) ** 2 + (f1 - c1) ** 2
    big = jnp.float32(jnp.inf)
    m1 = jnp.min(d2, axis=1, keepdims=True)
    d2a = jnp.where(d2 > m1, d2, big)
    m2 = jnp.min(d2a, axis=1, keepdims=True)
    d2b = jnp.where(d2a > m2, d2a, big)
    m3 = jnp.min(d2b, axis=1, keepdims=True)
    sel = (d2 <= m3).astype(jnp.float32)
    w = sel / jnp.maximum(d2, 1e-16)
    num = jnp.dot(w, cy_ref[...], preferred_element_type=jnp.float32)
    den = jnp.sum(w, axis=1, keepdims=True)
    out_ref[...] = num / den


_knn = pl.pallas_call(
    _knn_body,
    grid=(N_PAD // BN,),
    in_specs=[
        pl.BlockSpec((BN, 2), lambda i: (i, 0)),
        pl.BlockSpec((8, NC_PAD), lambda i: (0, 0)),
        pl.BlockSpec((NC_PAD, 8), lambda i: (0, 0)),
    ],
    out_specs=pl.BlockSpec((BN, 8), lambda i: (i, 0)),
    out_shape=jax.ShapeDtypeStruct((N_PAD, 8), jnp.float32),
    name="tc_knn",
)


def _make_finalize2(cp):
    def body(p1_ref, b1_ref, p2_ref, b2_ref, out_ref):
        a1 = jnp.tanh(p1_ref[0] + p1_ref[1] + b1_ref[...])
        a2 = jnp.tanh(p2_ref[0] + p2_ref[1] + b2_ref[...])
        out_ref[:, :cp] = jnp.maximum(a1, 0.0)
        out_ref[:, cp:] = jnp.maximum(a2, 0.0)

    return pl.pallas_call(
        body,
        grid=(N_PAD // BN,),
        in_specs=[
            pl.BlockSpec((2, BN, cp), lambda i: (0, i, 0)),
            pl.BlockSpec((1, cp), lambda i: (0, 0)),
            pl.BlockSpec((2, BN, cp), lambda i: (0, i, 0)),
            pl.BlockSpec((1, cp), lambda i: (0, 0)),
        ],
        out_specs=pl.BlockSpec((BN, 2 * cp), lambda i: (i, 0)),
        out_shape=jax.ShapeDtypeStruct((N_PAD, 2 * cp), jnp.float32),
        name="tc_finalize2",
    )


def _make_finalize1(cp):
    def body(p_ref, b_ref, out_ref):
        out_ref[...] = jnp.tanh(p_ref[0] + p_ref[1] + b_ref[...])

    return pl.pallas_call(
        body,
        grid=(N_PAD // BN,),
        in_specs=[
            pl.BlockSpec((2, BN, cp), lambda i: (0, i, 0)),
            pl.BlockSpec((1, cp), lambda i: (0, 0)),
        ],
        out_specs=pl.BlockSpec((BN, cp), lambda i: (i, 0)),
        out_shape=jax.ShapeDtypeStruct((N_PAD, cp), jnp.float32),
        name="tc_finalize1",
    )


def _pad2(a, rows, cols):
    return jnp.pad(a, ((0, rows - a.shape[0]), (0, cols - a.shape[1])))


def _prep_w(p, cin, cin_p, cout, cout_p):
    win = jnp.stack([p['Win'][:, h::3] for h in range(3)])
    win = jnp.pad(win, ((0, 0), (0, 8 - 6), (0, cin_p - cin)))
    bin_ = jnp.stack([p['bin'][h::3][None, :] for h in range(3)])
    bin_ = jnp.pad(bin_, ((0, 0), (0, 0), (0, cin_p - cin)))
    wout = jnp.stack([p['Wout'][h::3, :] for h in range(3)])
    wout = jnp.pad(wout, ((0, 0), (0, cin_p - cin), (0, cout_p - cout)))
    bout = jnp.pad(p['bout'][None, :], ((0, 0), (0, cout_p - cout)))
    return win, bin_, wout, bout


def kernel(x, sdf, edge_index, edge_indexA2, edge_attr, edge_attrA2,
           coarse_nodes, coarse_y, params):
    i32 = jnp.int32
    pe = E_PAD - N_EDGES

    def prep_edges(ei, ea):
        src = jnp.pad(ei[0].astype(i32), (0, pe))
        dst = jnp.pad(ei[1].astype(i32), (0, pe), constant_values=DUMMY)
        eap = _pad2(ea.astype(jnp.float32), E_PAD, 8)
        return src, dst, eap

    src1, dst1, ea1 = prep_edges(edge_index, edge_attr)
    src2, dst2, ea2 = prep_edges(edge_indexA2, edge_attrA2)

    g16 = _make_gather(16)
    g144 = _make_gather(144)
    g128 = _make_gather(128)
    sc2_64 = _make_scatter(2, 64)
    sc1_16 = _make_scatter(1, 16)
    d6 = _make_dense(16, 64)
    d131 = _make_dense(144, 64)
    d128 = _make_dense(128, 16)
    f2_64 = _make_finalize2(64)
    f1_16 = _make_finalize1(16)

    wp1 = _prep_w(params['pre0'][0], 6, 16, 64, 64)
    wp2 = _prep_w(params['pre0'][1], 6, 16, 64, 64)
    we1 = _prep_w(params['end0'][0], 131, 144, 64, 64)
    we2 = _prep_w(params['end0'][1], 131, 144, 64, 64)
    wl = _prep_w(params['end1'][0], 128, 128, 3, 16)

    t0 = _pad2(jnp.concatenate([x, sdf], axis=1), N_PAD, 16)
    xj1 = g16(t0, src1)
    xj2 = g16(t0, src2)
    c1 = d6(ea1, xj1, wp1[0], wp1[1], wp1[2])
    c2 = d6(ea2, xj2, wp2[0], wp2[1], wp2[2])
    p1, p2 = sc2_64(c1, dst1, c2, dst2)
    h0 = f2_64(p1, wp1[3], p2, wp2[3])

    fpos = _pad2(x[:, :2], N_PAD, 2)
    cpos = _pad2(coarse_nodes.T, 8, NC_PAD)
    cpos = cpos + jnp.pad(jnp.full((8, NC_PAD - N_COARSE), 1e30),
                          ((0, 0), (N_COARSE, 0)))
    cy = _pad2(coarse_y, NC_PAD, 8)
    yk = _knn(fpos, cpos, cy)

    z = jnp.concatenate([yk[:, :3], h0, jnp.zeros((N_PAD, 13), jnp.float32)],
                        axis=1)
    xj1 = g144(z, src1)
    xj2 = g144(z, src2)
    c1 = d131(ea1, xj1, we1[0], we1[1], we1[2])
    c2 = d131(ea2, xj2, we2[0], we2[1], we2[2])
    p1, p2 = sc2_64(c1, dst1, c2, dst2)
    h1 = f2_64(p1, we1[3], p2, we2[3])

    xj = g128(h1, src1)
    c = d128(ea1, xj, wl[0], wl[1], wl[2])
    (p,) = sc1_16(c, dst1)
    o = f1_16(p, wl[3])
    return o[:N_NODES, :3]


# speedup vs baseline: 4.1106x; 1.4122x over previous
import functools

import jax
import jax.numpy as jnp
from jax import lax
from jax.experimental import pallas as pl
from jax.experimental.pallas import tpu as pltpu
from jax.experimental.pallas import tpu_sc as plsc

N_NODES = 10000
N_EDGES = 160000
N_COARSE = 2000
N_PAD = 10240
E_PAD = 163840
NC_PAD = 2048
DUMMY = 10200

NCORES = 2
NSUB = 16
NW = NCORES * NSUB
CHUNK = 128
QT = E_PAD // NW
NCHUNK = QT // CHUNK
ROWS_T = N_PAD // NSUB


def _mesh():
    return plsc.VectorSubcoreMesh(core_axis_name="c", subcore_axis_name="s",
                                  num_cores=NCORES, num_subcores=NSUB)


_SC_PARAMS = pltpu.CompilerParams(use_tc_tiling_on_sc=False)


def _make_gather(cp):
    @functools.partial(
        pl.kernel,
        out_type=jax.ShapeDtypeStruct((E_PAD, cp), jnp.float32),
        mesh=_mesh(),
        scratch_types=[
            pltpu.VMEM((NCHUNK, CHUNK), jnp.int32),
            pltpu.VMEM((2, CHUNK, cp), jnp.float32),
            pltpu.SemaphoreType.DMA,
            pltpu.SemaphoreType.DMA,
        ],
        compiler_params=_SC_PARAMS,
        name=f"sc_gather_{cp}",
    )
    def gk(table, src2d, out, idxs, bufs, sem0, sem1):
        wid = lax.axis_index("s") * NCORES + lax.axis_index("c")
        base = wid * QT
        kbase = wid * NCHUNK
        sems = [sem0, sem1]
        pltpu.sync_copy(src2d.at[pl.ds(kbase, NCHUNK)], idxs)

        def start(k):
            b = k % 2
            return pltpu.async_copy(table.at[idxs.at[k]], bufs.at[b], sems[b])

        descs = [start(0), None]
        for k in range(NCHUNK):
            b = k % 2
            if k + 1 < NCHUNK:
                descs[1 - b] = start(k + 1)
            descs[b].wait()
            pltpu.sync_copy(bufs.at[b], out.at[pl.ds(base + k * CHUNK, CHUNK)])

    return gk


def _make_scatter(nsets, cp):
    out_types = [jax.ShapeDtypeStruct((NCORES, N_PAD, cp), jnp.float32)
                 for _ in range(nsets)]
    scratch = ([pltpu.VMEM_SHARED((N_PAD, cp), jnp.float32)
                for _ in range(nsets)]
               + [pltpu.VMEM((NCHUNK, CHUNK), jnp.int32)
                  for _ in range(nsets)]
               + [pltpu.VMEM((2, CHUNK, cp), jnp.float32),
                  pltpu.SemaphoreType.DMA,
                  pltpu.SemaphoreType.DMA])

    @functools.partial(pl.kernel, out_type=out_types, mesh=_mesh(),
                       scratch_types=scratch, compiler_params=_SC_PARAMS,
                       name=f"sc_scatter_{nsets}x{cp}")
    def sk(*refs):
        ins = refs[:2 * nsets]
        outs = refs[2 * nsets:3 * nsets]
        accs = refs[3 * nsets:4 * nsets]
        idxs = refs[4 * nsets:5 * nsets]
        bufs = refs[5 * nsets]
        sems = [refs[5 * nsets + 1], refs[5 * nsets + 2]]

        cid = lax.axis_index("c")
        sid = lax.axis_index("s")
        wid = sid * NCORES + cid
        base = wid * QT
        kbase = wid * NCHUNK
        zrow = sid * ROWS_T

        for i in range(nsets):
            pltpu.sync_copy(ins[2 * i + 1].at[pl.ds(kbase, NCHUNK)], idxs[i])

        zeros16 = jnp.zeros((16,), jnp.float32)

        def zb(i, carry):
            r = i // (cp // 16)
            j = i % (cp // 16)
            bufs[0, r, pl.ds(j * 16, 16)] = zeros16
            return carry

        lax.fori_loop(0, CHUNK * (cp // 16), zb, 0)
        for a in accs:
            def zs(t, carry, a=a):
                pltpu.sync_copy(bufs.at[0], a.at[pl.ds(zrow + t * CHUNK, CHUNK)])
                return carry
            lax.fori_loop(0, ROWS_T // CHUNK, zs, 0)
        plsc.subcore_barrier()

        chunks = [(i, k) for i in range(nsets) for k in range(NCHUNK)]

        def start(j):
            i, k = chunks[j]
            b = j % 2
            return pltpu.async_copy(
                ins[2 * i].at[pl.ds(base + k * CHUNK, CHUNK)],
                bufs.at[b], sems[b])

        descs = [start(0), None]
        for j in range(len(chunks)):
            i, k = chunks[j]
            b = j % 2
            if j + 1 < len(chunks):
                descs[1 - b] = start(j + 1)
            descs[b].wait()
            pltpu.sync_copy(bufs.at[b], accs[i].at[idxs[i].at[k]], add=True)

        plsc.subcore_barrier()
        for i in range(nsets):
            pltpu.sync_copy(accs[i].at[pl.ds(zrow, ROWS_T)],
                            outs[i].at[cid, pl.ds(zrow, ROWS_T)])

    return sk


def _make_dense(cin_p, cout_p, be=2048):
    grid = E_PAD // be

    def body(ea_ref, xj_ref, win_ref, bin_ref, wout_ref, out_ref):
        ea = ea_ref[...]
        xj = xj_ref[...]
        acc = jnp.zeros((be, cout_p), jnp.float32)
        for h in range(3):
            s = jnp.maximum(
                jnp.dot(ea, win_ref[h], preferred_element_type=jnp.float32)
                + bin_ref[h], 0.0)
            acc = acc + jnp.dot(s * xj, wout_ref[h],
                                preferred_element_type=jnp.float32)
        out_ref[...] = acc

    return pl.pallas_call(
        body,
        grid=(grid,),
        in_specs=[
            pl.BlockSpec((be, 8), lambda i: (i, 0)),
            pl.BlockSpec((be, cin_p), lambda i: (i, 0)),
            pl.BlockSpec((3, 8, cin_p), lambda i: (0, 0, 0)),
            pl.BlockSpec((3, 1, cin_p), lambda i: (0, 0, 0)),
            pl.BlockSpec((3, cin_p, cout_p), lambda i: (0, 0, 0)),
        ],
        out_specs=pl.BlockSpec((be, cout_p), lambda i: (i, 0)),
        out_shape=jax.ShapeDtypeStruct((E_PAD, cout_p), jnp.float32),
        name=f"tc_dense_{cin_p}_{cout_p}",
    )


BN = 512


def _knn_body(fp_ref, cpos_ref, cy_ref, out_ref):
    f0 = fp_ref[:, 0:1]
    f1 = fp_ref[:, 1:2]
    c0 = cpos_ref[0:1, :]
    c1 = cpos_ref[1:2, :]
    d2 = (f0 - c0) ** 2 + (f1 - c1) ** 2
    big = jnp.float32(jnp.inf)
    m1 = jnp.min(d2, axis=1, keepdims=True)
    d2a = jnp.where(d2 > m1, d2, big)
    m2 = jnp.min(d2a, axis=1, keepdims=True)
    d2b = jnp.where(d2a > m2, d2a, big)
    m3 = jnp.min(d2b, axis=1, keepdims=True)
    sel = (d2 <= m3).astype(jnp.float32)
    w = sel / jnp.maximum(d2, 1e-16)
    num = jnp.dot(w, cy_ref[...], preferred_element_type=jnp.float32)
    den = jnp.sum(w, axis=1, keepdims=True)
    out_ref[...] = num / den


_knn = pl.pallas_call(
    _knn_body,
    grid=(N_PAD // BN,),
    in_specs=[
        pl.BlockSpec((BN, 2), lambda i: (i, 0)),
        pl.BlockSpec((8, NC_PAD), lambda i: (0, 0)),
        pl.BlockSpec((NC_PAD, 8), lambda i: (0, 0)),
    ],
    out_specs=pl.BlockSpec((BN, 8), lambda i: (i, 0)),
    out_shape=jax.ShapeDtypeStruct((N_PAD, 8), jnp.float32),
    name="tc_knn",
)


def _make_finalize2(cp):
    def body(p1_ref, b1_ref, p2_ref, b2_ref, out_ref):
        a1 = jnp.tanh(p1_ref[0] + p1_ref[1] + b1_ref[...])
        a2 = jnp.tanh(p2_ref[0] + p2_ref[1] + b2_ref[...])
        out_ref[:, :cp] = jnp.maximum(a1, 0.0)
        out_ref[:, cp:] = jnp.maximum(a2, 0.0)

    return pl.pallas_call(
        body,
        grid=(N_PAD // BN,),
        in_specs=[
            pl.BlockSpec((2, BN, cp), lambda i: (0, i, 0)),
            pl.BlockSpec((1, cp), lambda i: (0, 0)),
            pl.BlockSpec((2, BN, cp), lambda i: (0, i, 0)),
            pl.BlockSpec((1, cp), lambda i: (0, 0)),
        ],
        out_specs=pl.BlockSpec((BN, 2 * cp), lambda i: (i, 0)),
        out_shape=jax.ShapeDtypeStruct((N_PAD, 2 * cp), jnp.float32),
        name="tc_finalize2",
    )


def _make_finalize1(cp):
    def body(p_ref, b_ref, out_ref):
        out_ref[...] = jnp.tanh(p_ref[0] + p_ref[1] + b_ref[...])

    return pl.pallas_call(
        body,
        grid=(N_PAD // BN,),
        in_specs=[
            pl.BlockSpec((2, BN, cp), lambda i: (0, i, 0)),
            pl.BlockSpec((1, cp), lambda i: (0, 0)),
        ],
        out_specs=pl.BlockSpec((BN, cp), lambda i: (i, 0)),
        out_shape=jax.ShapeDtypeStruct((N_PAD, cp), jnp.float32),
        name="tc_finalize1",
    )


def _pad2(a, rows, cols):
    return jnp.pad(a, ((0, rows - a.shape[0]), (0, cols - a.shape[1])))


def _prep_w(p, cin, cin_p, cout, cout_p):
    win = jnp.stack([p['Win'][:, h::3] for h in range(3)])
    win = jnp.pad(win, ((0, 0), (0, 8 - 6), (0, cin_p - cin)))
    bin_ = jnp.stack([p['bin'][h::3][None, :] for h in range(3)])
    bin_ = jnp.pad(bin_, ((0, 0), (0, 0), (0, cin_p - cin)))
    wout = jnp.stack([p['Wout'][h::3, :] for h in range(3)])
    wout = jnp.pad(wout, ((0, 0), (0, cin_p - cin), (0, cout_p - cout)))
    bout = jnp.pad(p['bout'][None, :], ((0, 0), (0, cout_p - cout)))
    return win, bin_, wout, bout


def kernel(x, sdf, edge_index, edge_indexA2, edge_attr, edge_attrA2,
           coarse_nodes, coarse_y, params):
    i32 = jnp.int32
    pe = E_PAD - N_EDGES

    pad_src = (jnp.arange(pe, dtype=i32) * 97) % N_NODES
    pad_dst = N_NODES + jnp.arange(pe, dtype=i32) % (N_PAD - N_NODES)

    def prep_edges(ei, ea):
        src = jnp.concatenate([ei[0].astype(i32), pad_src]
                              ).reshape(E_PAD // CHUNK, CHUNK)
        dst = jnp.concatenate([ei[1].astype(i32), pad_dst]
                              ).reshape(E_PAD // CHUNK, CHUNK)
        eap = _pad2(ea.astype(jnp.float32), E_PAD, 8)
        return src, dst, eap

    src1, dst1, ea1 = prep_edges(edge_index, edge_attr)
    src2, dst2, ea2 = prep_edges(edge_indexA2, edge_attrA2)

    g16 = _make_gather(16)
    g144 = _make_gather(144)
    g128 = _make_gather(128)
    sc2_64 = _make_scatter(2, 64)
    sc1_16 = _make_scatter(1, 16)
    d6 = _make_dense(16, 64)
    d131 = _make_dense(144, 64)
    d128 = _make_dense(128, 16)
    f2_64 = _make_finalize2(64)
    f1_16 = _make_finalize1(16)

    wp1 = _prep_w(params['pre0'][0], 6, 16, 64, 64)
    wp2 = _prep_w(params['pre0'][1], 6, 16, 64, 64)
    we1 = _prep_w(params['end0'][0], 131, 144, 64, 64)
    we2 = _prep_w(params['end0'][1], 131, 144, 64, 64)
    wl = _prep_w(params['end1'][0], 128, 128, 3, 16)

    t0 = _pad2(jnp.concatenate([x, sdf], axis=1), N_PAD, 16)
    xj1 = g16(t0, src1)
    xj2 = g16(t0, src2)
    c1 = d6(ea1, xj1, wp1[0], wp1[1], wp1[2])
    c2 = d6(ea2, xj2, wp2[0], wp2[1], wp2[2])
    p1, p2 = sc2_64(c1, dst1, c2, dst2)
    h0 = f2_64(p1, wp1[3], p2, wp2[3])

    fpos = _pad2(x[:, :2], N_PAD, 2)
    cpos = _pad2(coarse_nodes.T, 8, NC_PAD)
    cpos = cpos + jnp.pad(jnp.full((8, NC_PAD - N_COARSE), 1e30),
                          ((0, 0), (N_COARSE, 0)))
    cy = _pad2(coarse_y, NC_PAD, 8)
    yk = _knn(fpos, cpos, cy)

    z = jnp.concatenate([yk[:, :3], h0, jnp.zeros((N_PAD, 13), jnp.float32)],
                        axis=1)
    xj1 = g144(z, src1)
    xj2 = g144(z, src2)
    c1 = d131(ea1, xj1, we1[0], we1[1], we1[2])
    c2 = d131(ea2, xj2, we2[0], we2[1], we2[2])
    p1, p2 = sc2_64(c1, dst1, c2, dst2)
    h1 = f2_64(p1, we1[3], p2, we2[3])

    xj = g128(h1, src1)
    c = d128(ea1, xj, wl[0], wl[1], wl[2])
    (p,) = sc1_16(c, dst1)
    o = f1_16(p, wl[3])
    return o[:N_NODES, :3]


# speedup vs baseline: 4.5635x; 1.1102x over previous
import functools

import jax
import jax.numpy as jnp
from jax import lax
from jax.experimental import pallas as pl
from jax.experimental.pallas import tpu as pltpu
from jax.experimental.pallas import tpu_sc as plsc

N_NODES = 10000
N_EDGES = 160000
N_COARSE = 2000
N_PAD = 10240
E_PAD = 163840
NC_PAD = 2048
DUMMY = 10200

NCORES = 2
NSUB = 16
NW = NCORES * NSUB
CHUNK = 128
QT = E_PAD // NW
NCHUNK = QT // CHUNK
ROWS_T = N_PAD // NSUB


def _mesh():
    return plsc.VectorSubcoreMesh(core_axis_name="c", subcore_axis_name="s",
                                  num_cores=NCORES, num_subcores=NSUB)


_SC_PARAMS = pltpu.CompilerParams(use_tc_tiling_on_sc=False)


def _make_gather(cp):
    @functools.partial(
        pl.kernel,
        out_type=jax.ShapeDtypeStruct((E_PAD, cp), jnp.float32),
        mesh=_mesh(),
        scratch_types=[
            pltpu.VMEM((NCHUNK, CHUNK), jnp.int32),
            pltpu.VMEM((2, CHUNK, cp), jnp.float32),
            pltpu.SemaphoreType.DMA,
            pltpu.SemaphoreType.DMA,
        ],
        compiler_params=_SC_PARAMS,
        name=f"sc_gather_{cp}",
    )
    def gk(table, src2d, out, idxs, bufs, sem0, sem1):
        wid = lax.axis_index("s") * NCORES + lax.axis_index("c")
        base = wid * QT
        kbase = wid * NCHUNK
        sems = [sem0, sem1]
        pltpu.sync_copy(src2d.at[pl.ds(kbase, NCHUNK)], idxs)

        def start(k):
            b = k % 2
            return pltpu.async_copy(table.at[idxs.at[k]], bufs.at[b], sems[b])

        descs = [start(0), None]
        for k in range(NCHUNK):
            b = k % 2
            if k + 1 < NCHUNK:
                descs[1 - b] = start(k + 1)
            descs[b].wait()
            pltpu.sync_copy(bufs.at[b], out.at[pl.ds(base + k * CHUNK, CHUNK)])

    return gk


def _make_scatter(nsets, cp):
    out_types = [jax.ShapeDtypeStruct((NCORES, N_PAD, cp), jnp.float32)
                 for _ in range(nsets)]
    scratch = ([pltpu.VMEM_SHARED((N_PAD, cp), jnp.float32)
                for _ in range(nsets)]
               + [pltpu.VMEM((NCHUNK, CHUNK), jnp.int32)
                  for _ in range(nsets)]
               + [pltpu.VMEM((2, CHUNK, cp), jnp.float32),
                  pltpu.SemaphoreType.DMA,
                  pltpu.SemaphoreType.DMA])

    @functools.partial(pl.kernel, out_type=out_types, mesh=_mesh(),
                       scratch_types=scratch, compiler_params=_SC_PARAMS,
                       name=f"sc_scatter_{nsets}x{cp}")
    def sk(*refs):
        ins = refs[:2 * nsets]
        outs = refs[2 * nsets:3 * nsets]
        accs = refs[3 * nsets:4 * nsets]
        idxs = refs[4 * nsets:5 * nsets]
        bufs = refs[5 * nsets]
        sems = [refs[5 * nsets + 1], refs[5 * nsets + 2]]

        cid = lax.axis_index("c")
        sid = lax.axis_index("s")
        wid = sid * NCORES + cid
        base = wid * QT
        kbase = wid * NCHUNK
        zrow = sid * ROWS_T

        for i in range(nsets):
            pltpu.sync_copy(ins[2 * i + 1].at[pl.ds(kbase, NCHUNK)], idxs[i])

        zeros16 = jnp.zeros((16,), jnp.float32)

        def zb(i, carry):
            r = i // (cp // 16)
            j = i % (cp // 16)
            bufs[0, r, pl.ds(j * 16, 16)] = zeros16
            return carry

        lax.fori_loop(0, CHUNK * (cp // 16), zb, 0)
        for a in accs:
            def zs(t, carry, a=a):
                pltpu.sync_copy(bufs.at[0], a.at[pl.ds(zrow + t * CHUNK, CHUNK)])
                return carry
            lax.fori_loop(0, ROWS_T // CHUNK, zs, 0)
        plsc.subcore_barrier()

        chunks = [(i, k) for i in range(nsets) for k in range(NCHUNK)]

        def start(j):
            i, k = chunks[j]
            b = j % 2
            return pltpu.async_copy(
                ins[2 * i].at[pl.ds(base + k * CHUNK, CHUNK)],
                bufs.at[b], sems[b])

        descs = [start(0), None]
        for j in range(len(chunks)):
            i, k = chunks[j]
            b = j % 2
            if j + 1 < len(chunks):
                descs[1 - b] = start(j + 1)
            descs[b].wait()
            pltpu.sync_copy(bufs.at[b], accs[i].at[idxs[i].at[k]], add=True)

        plsc.subcore_barrier()
        for i in range(nsets):
            pltpu.sync_copy(accs[i].at[pl.ds(zrow, ROWS_T)],
                            outs[i].at[cid, pl.ds(zrow, ROWS_T)])

    return sk


def _make_dense(cin_p, cout_p, be=2048):
    grid = E_PAD // be

    def body(ea_ref, xj_ref, win_ref, bin_ref, wout_ref, out_ref):
        eaT = ea_ref[...]
        xj = xj_ref[...]
        acc = jnp.zeros((be, cout_p), jnp.float32)
        for h in range(3):
            s = jnp.maximum(
                lax.dot_general(eaT, win_ref[h], (((0,), (0,)), ((), ())),
                                preferred_element_type=jnp.float32)
                + bin_ref[h], 0.0)
            acc = acc + jnp.dot(s * xj, wout_ref[h],
                                preferred_element_type=jnp.float32)
        out_ref[...] = acc

    return pl.pallas_call(
        body,
        grid=(grid,),
        in_specs=[
            pl.BlockSpec((8, be), lambda i: (0, i)),
            pl.BlockSpec((be, cin_p), lambda i: (i, 0)),
            pl.BlockSpec((3, 8, cin_p), lambda i: (0, 0, 0)),
            pl.BlockSpec((3, 1, cin_p), lambda i: (0, 0, 0)),
            pl.BlockSpec((3, cin_p, cout_p), lambda i: (0, 0, 0)),
        ],
        out_specs=pl.BlockSpec((be, cout_p), lambda i: (i, 0)),
        out_shape=jax.ShapeDtypeStruct((E_PAD, cout_p), jnp.float32),
        name=f"tc_dense_{cin_p}_{cout_p}",
    )


BN = 512


def _knn_body(fp_ref, cpos_ref, cy_ref, out_ref):
    f0 = fp_ref[:, 0:1]
    f1 = fp_ref[:, 1:2]
    c0 = cpos_ref[0:1, :]
    c1 = cpos_ref[1:2, :]
    d2 = (f0 - c0) ** 2 + (f1 - c1) ** 2
    big = jnp.float32(jnp.inf)
    m1 = jnp.min(d2, axis=1, keepdims=True)
    d2a = jnp.where(d2 > m1, d2, big)
    m2 = jnp.min(d2a, axis=1, keepdims=True)
    d2b = jnp.where(d2a > m2, d2a, big)
    m3 = jnp.min(d2b, axis=1, keepdims=True)
    sel = (d2 <= m3).astype(jnp.float32)
    w = sel / jnp.maximum(d2, 1e-16)
    num = jnp.dot(w, cy_ref[...], preferred_element_type=jnp.float32)
    den = jnp.sum(w, axis=1, keepdims=True)
    out_ref[...] = num / den


_knn = pl.pallas_call(
    _knn_body,
    grid=(N_PAD // BN,),
    in_specs=[
        pl.BlockSpec((BN, 2), lambda i: (i, 0)),
        pl.BlockSpec((8, NC_PAD), lambda i: (0, 0)),
        pl.BlockSpec((NC_PAD, 8), lambda i: (0, 0)),
    ],
    out_specs=pl.BlockSpec((BN, 8), lambda i: (i, 0)),
    out_shape=jax.ShapeDtypeStruct((N_PAD, 8), jnp.float32),
    name="tc_knn",
)


def _make_finalize2(cp):
    def body(p1_ref, b1_ref, p2_ref, b2_ref, out_ref):
        a1 = jnp.tanh(p1_ref[0] + p1_ref[1] + b1_ref[...])
        a2 = jnp.tanh(p2_ref[0] + p2_ref[1] + b2_ref[...])
        out_ref[:, :cp] = jnp.maximum(a1, 0.0)
        out_ref[:, cp:] = jnp.maximum(a2, 0.0)

    return pl.pallas_call(
        body,
        grid=(N_PAD // BN,),
        in_specs=[
            pl.BlockSpec((2, BN, cp), lambda i: (0, i, 0)),
            pl.BlockSpec((1, cp), lambda i: (0, 0)),
            pl.BlockSpec((2, BN, cp), lambda i: (0, i, 0)),
            pl.BlockSpec((1, cp), lambda i: (0, 0)),
        ],
        out_specs=pl.BlockSpec((BN, 2 * cp), lambda i: (i, 0)),
        out_shape=jax.ShapeDtypeStruct((N_PAD, 2 * cp), jnp.float32),
        name="tc_finalize2",
    )


def _make_finalize1(cp):
    def body(p_ref, b_ref, out_ref):
        out_ref[...] = jnp.tanh(p_ref[0] + p_ref[1] + b_ref[...])

    return pl.pallas_call(
        body,
        grid=(N_PAD // BN,),
        in_specs=[
            pl.BlockSpec((2, BN, cp), lambda i: (0, i, 0)),
            pl.BlockSpec((1, cp), lambda i: (0, 0)),
        ],
        out_specs=pl.BlockSpec((BN, cp), lambda i: (i, 0)),
        out_shape=jax.ShapeDtypeStruct((N_PAD, cp), jnp.float32),
        name="tc_finalize1",
    )


def _pad2(a, rows, cols):
    return jnp.pad(a, ((0, rows - a.shape[0]), (0, cols - a.shape[1])))


def _prep_w(p, cin, cin_p, cout, cout_p):
    win = jnp.stack([p['Win'][:, h::3] for h in range(3)])
    win = jnp.pad(win, ((0, 0), (0, 8 - 6), (0, cin_p - cin)))
    bin_ = jnp.stack([p['bin'][h::3][None, :] for h in range(3)])
    bin_ = jnp.pad(bin_, ((0, 0), (0, 0), (0, cin_p - cin)))
    wout = jnp.stack([p['Wout'][h::3, :] for h in range(3)])
    wout = jnp.pad(wout, ((0, 0), (0, cin_p - cin), (0, cout_p - cout)))
    bout = jnp.pad(p['bout'][None, :], ((0, 0), (0, cout_p - cout)))
    return win, bin_, wout, bout


def kernel(x, sdf, edge_index, edge_indexA2, edge_attr, edge_attrA2,
           coarse_nodes, coarse_y, params):
    i32 = jnp.int32
    pe = E_PAD - N_EDGES

    pad_src = (jnp.arange(pe, dtype=i32) * 97) % N_NODES
    pad_dst = N_NODES + jnp.arange(pe, dtype=i32) % (N_PAD - N_NODES)

    def prep_edges(ei, ea):
        src = jnp.concatenate([ei[0].astype(i32), pad_src]
                              ).reshape(E_PAD // CHUNK, CHUNK)
        dst = jnp.concatenate([ei[1].astype(i32), pad_dst]
                              ).reshape(E_PAD // CHUNK, CHUNK)
        eaT = _pad2(ea.T.astype(jnp.float32), 8, E_PAD)
        return src, dst, eaT

    src1, dst1, ea1 = prep_edges(edge_index, edge_attr)
    src2, dst2, ea2 = prep_edges(edge_indexA2, edge_attrA2)

    g16 = _make_gather(16)
    g144 = _make_gather(144)
    g128 = _make_gather(128)
    sc2_64 = _make_scatter(2, 64)
    sc1_16 = _make_scatter(1, 16)
    d6 = _make_dense(16, 64)
    d131 = _make_dense(144, 64)
    d128 = _make_dense(128, 16)
    f2_64 = _make_finalize2(64)
    f1_16 = _make_finalize1(16)

    wp1 = _prep_w(params['pre0'][0], 6, 16, 64, 64)
    wp2 = _prep_w(params['pre0'][1], 6, 16, 64, 64)
    we1 = _prep_w(params['end0'][0], 131, 144, 64, 64)
    we2 = _prep_w(params['end0'][1], 131, 144, 64, 64)
    wl = _prep_w(params['end1'][0], 128, 128, 3, 16)

    t0 = _pad2(jnp.concatenate([x, sdf], axis=1), N_PAD, 16)
    xj1 = g16(t0, src1)
    xj2 = g16(t0, src2)
    c1 = d6(ea1, xj1, wp1[0], wp1[1], wp1[2])
    c2 = d6(ea2, xj2, wp2[0], wp2[1], wp2[2])
    p1, p2 = sc2_64(c1, dst1, c2, dst2)
    h0 = f2_64(p1, wp1[3], p2, wp2[3])

    fpos = _pad2(x[:, :2], N_PAD, 2)
    cpos = _pad2(coarse_nodes.T, 8, NC_PAD)
    cpos = cpos + jnp.pad(jnp.full((8, NC_PAD - N_COARSE), 1e30),
                          ((0, 0), (N_COARSE, 0)))
    cy = _pad2(coarse_y, NC_PAD, 8)
    yk = _knn(fpos, cpos, cy)

    z = jnp.concatenate([yk[:, :3], h0, jnp.zeros((N_PAD, 13), jnp.float32)],
                        axis=1)
    xj1 = g144(z, src1)
    xj2 = g144(z, src2)
    c1 = d131(ea1, xj1, we1[0], we1[1], we1[2])
    c2 = d131(ea2, xj2, we2[0], we2[1], we2[2])
    p1, p2 = sc2_64(c1, dst1, c2, dst2)
    h1 = f2_64(p1, we1[3], p2, we2[3])

    xj = g128(h1, src1)
    c = d128(ea1, xj, wl[0], wl[1], wl[2])
    (p,) = sc1_16(c, dst1)
    o = f1_16(p, wl[3])
    return o[:N_NODES, :3]


# speedup vs baseline: 6.7809x; 1.4859x over previous
import functools

import jax
import jax.numpy as jnp
from jax import lax
from jax.experimental import pallas as pl
from jax.experimental.pallas import tpu as pltpu
from jax.experimental.pallas import tpu_sc as plsc

N_NODES = 10000
N_EDGES = 160000
N_COARSE = 2000
N_PAD = 10240
E_PAD = 163840
NC_PAD = 2048
W = 128

NCORES = 2
NSUB = 16
NW = NCORES * NSUB
CHUNK = 128
QT = E_PAD // NW
NCHUNK = QT // CHUNK
ROWS_T = N_PAD // NSUB


def _mesh():
    return plsc.VectorSubcoreMesh(core_axis_name="c", subcore_axis_name="s",
                                  num_cores=NCORES, num_subcores=NSUB)


_SC_PARAMS = pltpu.CompilerParams(use_tc_tiling_on_sc=False)


@functools.partial(
    pl.kernel,
    out_type=jax.ShapeDtypeStruct((E_PAD, W), jnp.float32),
    mesh=_mesh(),
    scratch_types=[
        pltpu.VMEM((NCHUNK, CHUNK), jnp.int32),
        pltpu.VMEM((2, CHUNK, W), jnp.float32),
        pltpu.SemaphoreType.DMA,
        pltpu.SemaphoreType.DMA,
    ],
    compiler_params=_SC_PARAMS,
    name="sc_gather_128",
)
def _g128(table, src2d, out, idxs, bufs, sem0, sem1):
    wid = lax.axis_index("s") * NCORES + lax.axis_index("c")
    base = wid * QT
    kbase = wid * NCHUNK
    sems = [sem0, sem1]
    pltpu.sync_copy(src2d.at[pl.ds(kbase, NCHUNK)], idxs)

    def start(k):
        b = k % 2
        return pltpu.async_copy(table.at[idxs.at[k]], bufs.at[b], sems[b])

    descs = [start(0), None]
    for k in range(NCHUNK):
        b = k % 2
        if k + 1 < NCHUNK:
            descs[1 - b] = start(k + 1)
        descs[b].wait()
        pltpu.sync_copy(bufs.at[b], out.at[pl.ds(base + k * CHUNK, CHUNK)])


def _make_scatter(nsets, cp):
    out_types = [jax.ShapeDtypeStruct((N_PAD, NCORES * cp), jnp.float32)
                 for _ in range(nsets)]
    scratch = ([pltpu.VMEM_SHARED((N_PAD, cp), jnp.float32)
                for _ in range(nsets)]
               + [pltpu.VMEM((NCHUNK, CHUNK), jnp.int32)
                  for _ in range(nsets)]
               + [pltpu.VMEM((2, CHUNK, cp), jnp.float32),
                  pltpu.SemaphoreType.DMA,
                  pltpu.SemaphoreType.DMA])

    @functools.partial(pl.kernel, out_type=out_types, mesh=_mesh(),
                       scratch_types=scratch, compiler_params=_SC_PARAMS,
                       name=f"sc_scatter_{nsets}x{cp}")
    def sk(*refs):
        ins = refs[:2 * nsets]
        outs = refs[2 * nsets:3 * nsets]
        accs = refs[3 * nsets:4 * nsets]
        idxs = refs[4 * nsets:5 * nsets]
        bufs = refs[5 * nsets]
        sems = [refs[5 * nsets + 1], refs[5 * nsets + 2]]

        cid = lax.axis_index("c")
        sid = lax.axis_index("s")
        wid = sid * NCORES + cid
        base = wid * QT
        kbase = wid * NCHUNK
        zrow = sid * ROWS_T

        for i in range(nsets):
            pltpu.sync_copy(ins[2 * i + 1].at[pl.ds(kbase, NCHUNK)], idxs[i])

        zeros16 = jnp.zeros((16,), jnp.float32)

        def zb(i, carry):
            r = i // (cp // 16)
            j = i % (cp // 16)
            bufs[0, r, pl.ds(j * 16, 16)] = zeros16
            return carry

        lax.fori_loop(0, CHUNK * (cp // 16), zb, 0)
        for a in accs:
            def zs(t, carry, a=a):
                pltpu.sync_copy(bufs.at[0], a.at[pl.ds(zrow + t * CHUNK, CHUNK)])
                return carry
            lax.fori_loop(0, ROWS_T // CHUNK, zs, 0)
        plsc.subcore_barrier()

        chunks = [(i, k) for i in range(nsets) for k in range(NCHUNK)]

        def start(j):
            i, k = chunks[j]
            b = j % 2
            return pltpu.async_copy(
                ins[2 * i].at[pl.ds(base + k * CHUNK, CHUNK), pl.ds(0, cp)],
                bufs.at[b], sems[b])

        descs = [start(0), None]
        for j in range(len(chunks)):
            i, k = chunks[j]
            b = j % 2
            if j + 1 < len(chunks):
                descs[1 - b] = start(j + 1)
            descs[b].wait()
            pltpu.sync_copy(bufs.at[b], accs[i].at[idxs[i].at[k]], add=True)

        plsc.subcore_barrier()
        for i in range(nsets):
            pltpu.sync_copy(accs[i].at[pl.ds(zrow, ROWS_T)],
                            outs[i].at[pl.ds(zrow, ROWS_T),
                                       pl.ds(cid * cp, cp)])

    return sk


BE = 2048


def _make_dense_pre():
    def body(ea_ref, xj_ref, win_ref, bin_ref, wout_ref, out_ref):
        eaT = ea_ref[...]
        xj = xj_ref[:, :16]
        acc = jnp.zeros((BE, 64), jnp.float32)
        for h in range(3):
            s = jnp.maximum(
                lax.dot_general(eaT, win_ref[h], (((0,), (0,)), ((), ())),
                                preferred_element_type=jnp.float32)
                + bin_ref[h], 0.0)
            acc = acc + jnp.dot(s * xj, wout_ref[h],
                                preferred_element_type=jnp.float32)
        out_ref[:, :64] = acc
        out_ref[:, 64:] = jnp.zeros((BE, W - 64), jnp.float32)

    return pl.pallas_call(
        body,
        grid=(E_PAD // BE,),
        in_specs=[
            pl.BlockSpec((8, BE), lambda i: (0, i)),
            pl.BlockSpec((BE, W), lambda i: (i, 0)),
            pl.BlockSpec((3, 8, 16), lambda i: (0, 0, 0)),
            pl.BlockSpec((3, 1, 16), lambda i: (0, 0, 0)),
            pl.BlockSpec((3, 16, 64), lambda i: (0, 0, 0)),
        ],
        out_specs=pl.BlockSpec((BE, W), lambda i: (i, 0)),
        out_shape=jax.ShapeDtypeStruct((E_PAD, W), jnp.float32),
        name="tc_dense_pre",
    )


def _make_dense_end0():
    def body(ea_ref, xjh_ref, xjy_ref, winy_ref, biny_ref, wouty_ref,
             winh_ref, binh_ref, wouth_ref, out_ref):
        eaT = ea_ref[...]
        xjh = xjh_ref[...]
        xjy = xjy_ref[:, :16]
        acc = jnp.zeros((BE, 64), jnp.float32)
        for h in range(3):
            sy = jnp.maximum(
                lax.dot_general(eaT, winy_ref[h], (((0,), (0,)), ((), ())),
                                preferred_element_type=jnp.float32)
                + biny_ref[h], 0.0)
            acc = acc + jnp.dot(sy * xjy, wouty_ref[h],
                                preferred_element_type=jnp.float32)
            sh = jnp.maximum(
                lax.dot_general(eaT, winh_ref[h], (((0,), (0,)), ((), ())),
                                preferred_element_type=jnp.float32)
                + binh_ref[h], 0.0)
            acc = acc + jnp.dot(sh * xjh, wouth_ref[h],
                                preferred_element_type=jnp.float32)
        out_ref[:, :64] = acc
        out_ref[:, 64:] = jnp.zeros((BE, W - 64), jnp.float32)

    return pl.pallas_call(
        body,
        grid=(E_PAD // BE,),
        in_specs=[
            pl.BlockSpec((8, BE), lambda i: (0, i)),
            pl.BlockSpec((BE, W), lambda i: (i, 0)),
            pl.BlockSpec((BE, W), lambda i: (i, 0)),
            pl.BlockSpec((3, 8, 16), lambda i: (0, 0, 0)),
            pl.BlockSpec((3, 1, 16), lambda i: (0, 0, 0)),
            pl.BlockSpec((3, 16, 64), lambda i: (0, 0, 0)),
            pl.BlockSpec((3, 8, W), lambda i: (0, 0, 0)),
            pl.BlockSpec((3, 1, W), lambda i: (0, 0, 0)),
            pl.BlockSpec((3, W, 64), lambda i: (0, 0, 0)),
        ],
        out_specs=pl.BlockSpec((BE, W), lambda i: (i, 0)),
        out_shape=jax.ShapeDtypeStruct((E_PAD, W), jnp.float32),
        name="tc_dense_end0",
    )


def _make_dense_end1():
    def body(ea_ref, xj_ref, win_ref, bin_ref, wout_ref, out_ref):
        eaT = ea_ref[...]
        xj = xj_ref[...]
        acc = jnp.zeros((BE, 16), jnp.float32)
        for h in range(3):
            s = jnp.maximum(
                lax.dot_general(eaT, win_ref[h], (((0,), (0,)), ((), ())),
                                preferred_element_type=jnp.float32)
                + bin_ref[h], 0.0)
            acc = acc + jnp.dot(s * xj, wout_ref[h],
                                preferred_element_type=jnp.float32)
        out_ref[:, :16] = acc
        out_ref[:, 16:] = jnp.zeros((BE, W - 16), jnp.float32)

    return pl.pallas_call(
        body,
        grid=(E_PAD // BE,),
        in_specs=[
            pl.BlockSpec((8, BE), lambda i: (0, i)),
            pl.BlockSpec((BE, W), lambda i: (i, 0)),
            pl.BlockSpec((3, 8, W), lambda i: (0, 0, 0)),
            pl.BlockSpec((3, 1, W), lambda i: (0, 0, 0)),
            pl.BlockSpec((3, W, 16), lambda i: (0, 0, 0)),
        ],
        out_specs=pl.BlockSpec((BE, W), lambda i: (i, 0)),
        out_shape=jax.ShapeDtypeStruct((E_PAD, W), jnp.float32),
        name="tc_dense_end1",
    )


BN = 512


def _knn_body(fp_ref, cpos_ref, cy_ref, out_ref):
    f0 = fp_ref[:, 0:1]
    f1 = fp_ref[:, 1:2]
    c0 = cpos_ref[0:1, :]
    c1 = cpos_ref[1:2, :]
    d2 = (f0 - c0) ** 2 + (f1 - c1) ** 2
    big = jnp.float32(jnp.inf)
    m1 = jnp.min(d2, axis=1, keepdims=True)
    d2a = jnp.where(d2 > m1, d2, big)
    m2 = jnp.min(d2a, axis=1, keepdims=True)
    d2b = jnp.where(d2a > m2, d2a, big)
    m3 = jnp.min(d2b, axis=1, keepdims=True)
    sel = (d2 <= m3).astype(jnp.float32)
    w = sel / jnp.maximum(d2, 1e-16)
    num = jnp.dot(w, cy_ref[...], preferred_element_type=jnp.float32)
    den = jnp.sum(w, axis=1, keepdims=True)
    out_ref[...] = num / den


_knn = pl.pallas_call(
    _knn_body,
    grid=(N_PAD // BN,),
    in_specs=[
        pl.BlockSpec((BN, 2), lambda i: (i, 0)),
        pl.BlockSpec((8, NC_PAD), lambda i: (0, 0)),
        pl.BlockSpec((NC_PAD, 8), lambda i: (0, 0)),
    ],
    out_specs=pl.BlockSpec((BN, 8), lambda i: (i, 0)),
    out_shape=jax.ShapeDtypeStruct((N_PAD, 8), jnp.float32),
    name="tc_knn",
)


def _finalize2_body(p1_ref, b1_ref, p2_ref, b2_ref, out_ref):
    a1 = jnp.tanh(p1_ref[:, :64] + p1_ref[:, 64:] + b1_ref[...])
    a2 = jnp.tanh(p2_ref[:, :64] + p2_ref[:, 64:] + b2_ref[...])
    out_ref[:, :64] = jnp.maximum(a1, 0.0)
    out_ref[:, 64:] = jnp.maximum(a2, 0.0)


_finalize2 = pl.pallas_call(
    _finalize2_body,
    grid=(N_PAD // BN,),
    in_specs=[
        pl.BlockSpec((BN, W), lambda i: (i, 0)),
        pl.BlockSpec((1, 64), lambda i: (0, 0)),
        pl.BlockSpec((BN, W), lambda i: (i, 0)),
        pl.BlockSpec((1, 64), lambda i: (0, 0)),
    ],
    out_specs=pl.BlockSpec((BN, W), lambda i: (i, 0)),
    out_shape=jax.ShapeDtypeStruct((N_PAD, W), jnp.float32),
    name="tc_finalize2",
)


def _finalize1_body(p_ref, b_ref, out_ref):
    out_ref[...] = jnp.tanh(p_ref[:, :16] + p_ref[:, 16:] + b_ref[...])


_finalize1 = pl.pallas_call(
    _finalize1_body,
    grid=(N_PAD // BN,),
    in_specs=[
        pl.BlockSpec((BN, 32), lambda i: (i, 0)),
        pl.BlockSpec((1, 16), lambda i: (0, 0)),
    ],
    out_specs=pl.BlockSpec((BN, 16), lambda i: (i, 0)),
    out_shape=jax.ShapeDtypeStruct((N_PAD, 16), jnp.float32),
    name="tc_finalize1",
)


def _pad2(a, rows, cols):
    return jnp.pad(a, ((0, rows - a.shape[0]), (0, cols - a.shape[1])))


def _split_w(p, cin):
    win = jnp.stack([p['Win'][:, h::3] for h in range(3)])
    bin_ = jnp.stack([p['bin'][h::3][None, :] for h in range(3)])
    wout = jnp.stack([p['Wout'][h::3, :] for h in range(3)])
    assert win.shape[2] == cin
    return win, bin_, wout


def _prep_pre(p):
    win, bin_, wout = _split_w(p, 6)
    return (jnp.pad(win, ((0, 0), (0, 2), (0, 10))),
            jnp.pad(bin_, ((0, 0), (0, 0), (0, 10))),
            jnp.pad(wout, ((0, 0), (0, 10), (0, 0))),
            p['bout'][None, :])


def _prep_end0(p):
    win, bin_, wout = _split_w(p, 131)
    winy = jnp.pad(win[:, :, :3], ((0, 0), (0, 2), (0, 13)))
    biny = jnp.pad(bin_[:, :, :3], ((0, 0), (0, 0), (0, 13)))
    wouty = jnp.pad(wout[:, :3, :], ((0, 0), (0, 13), (0, 0)))
    winh = jnp.pad(win[:, :, 3:], ((0, 0), (0, 2), (0, 0)))
    binh = bin_[:, :, 3:]
    wouth = wout[:, 3:, :]
    return winy, biny, wouty, winh, binh, wouth, p['bout'][None, :]


def _prep_end1(p):
    win, bin_, wout = _split_w(p, 128)
    return (jnp.pad(win, ((0, 0), (0, 2), (0, 0))),
            bin_,
            jnp.pad(wout, ((0, 0), (0, 0), (0, 16 - wout.shape[2]))),
            jnp.pad(p['bout'][None, :], ((0, 0), (0, 13))))


def kernel(x, sdf, edge_index, edge_indexA2, edge_attr, edge_attrA2,
           coarse_nodes, coarse_y, params):
    i32 = jnp.int32
    pe = E_PAD - N_EDGES

    pad_src = (jnp.arange(pe, dtype=i32) * 97) % N_NODES
    pad_dst = N_NODES + jnp.arange(pe, dtype=i32) % (N_PAD - N_NODES)

    def prep_edges(ei, ea):
        src = jnp.concatenate([ei[0].astype(i32), pad_src]
                              ).reshape(E_PAD // CHUNK, CHUNK)
        dst = jnp.concatenate([ei[1].astype(i32), pad_dst]
                              ).reshape(E_PAD // CHUNK, CHUNK)
        eaT = _pad2(ea.T.astype(jnp.float32), 8, E_PAD)
        return src, dst, eaT

    src1, dst1, ea1 = prep_edges(edge_index, edge_attr)
    src2, dst2, ea2 = prep_edges(edge_indexA2, edge_attrA2)

    sc2_64 = _make_scatter(2, 64)
    sc1_16 = _make_scatter(1, 16)
    d_pre = _make_dense_pre()
    d_end0 = _make_dense_end0()
    d_end1 = _make_dense_end1()

    wp1 = _prep_pre(params['pre0'][0])
    wp2 = _prep_pre(params['pre0'][1])
    we1 = _prep_end0(params['end0'][0])
    we2 = _prep_end0(params['end0'][1])
    wl = _prep_end1(params['end1'][0])

    t0 = _pad2(jnp.concatenate([x, sdf], axis=1), N_PAD, W)
    xj1 = _g128(t0, src1)
    xj2 = _g128(t0, src2)
    c1 = d_pre(ea1, xj1, wp1[0], wp1[1], wp1[2])
    c2 = d_pre(ea2, xj2, wp2[0], wp2[1], wp2[2])
    p1, p2 = sc2_64(c1, dst1, c2, dst2)
    h0 = _finalize2(p1, wp1[3], p2, wp2[3])

    fpos = _pad2(x[:, :2], N_PAD, 2)
    cpos = _pad2(coarse_nodes.T, 8, NC_PAD)
    cpos = cpos + jnp.pad(jnp.full((8, NC_PAD - N_COARSE), 1e30),
                          ((0, 0), (N_COARSE, 0)))
    cy = _pad2(coarse_y, NC_PAD, 8)
    yk = _knn(fpos, cpos, cy)
    yky = _pad2(yk, N_PAD, W)

    xh1 = _g128(h0, src1)
    xy1 = _g128(yky, src1)
    c1 = d_end0(ea1, xh1, xy1, *we1[:6])
    xh2 = _g128(h0, src2)
    xy2 = _g128(yky, src2)
    c2 = d_end0(ea2, xh2, xy2, *we2[:6])
    p1, p2 = sc2_64(c1, dst1, c2, dst2)
    h1 = _finalize2(p1, we1[6], p2, we2[6])

    xj = _g128(h1, src1)
    c = d_end1(ea1, xj, wl[0], wl[1], wl[2])
    (p,) = sc1_16(c, dst1)
    o = _finalize1(p, wl[3])
    return o[:N_NODES, :3]


# speedup vs baseline: 7.2747x; 1.0728x over previous
import functools

import jax
import jax.numpy as jnp
from jax import lax
from jax.experimental import pallas as pl
from jax.experimental.pallas import tpu as pltpu
from jax.experimental.pallas import tpu_sc as plsc

N_NODES = 10000
N_EDGES = 160000
N_COARSE = 2000
N_PAD = 10240
E_PAD = 163840
NC_PAD = 2048
W = 128

NCORES = 2
NSUB = 16
NW = NCORES * NSUB
CHUNK = 128
QT = E_PAD // NW
NCHUNK = QT // CHUNK
ROWS_T = N_PAD // NSUB


def _mesh():
    return plsc.VectorSubcoreMesh(core_axis_name="c", subcore_axis_name="s",
                                  num_cores=NCORES, num_subcores=NSUB)


_SC_PARAMS = pltpu.CompilerParams(use_tc_tiling_on_sc=False)


@functools.partial(
    pl.kernel,
    out_type=jax.ShapeDtypeStruct((E_PAD, W), jnp.float32),
    mesh=_mesh(),
    scratch_types=[
        pltpu.VMEM((NCHUNK, CHUNK), jnp.int32),
        pltpu.VMEM((2, CHUNK, W), jnp.float32),
        pltpu.SemaphoreType.DMA,
        pltpu.SemaphoreType.DMA,
    ],
    compiler_params=_SC_PARAMS,
    name="sc_gather_128",
)
def _g128(table, src2d, out, idxs, bufs, sem0, sem1):
    wid = lax.axis_index("s") * NCORES + lax.axis_index("c")
    base = wid * QT
    kbase = wid * NCHUNK
    sems = [sem0, sem1]
    pltpu.sync_copy(src2d.at[pl.ds(kbase, NCHUNK)], idxs)

    def start(k):
        b = k % 2
        return pltpu.async_copy(table.at[idxs.at[k]], bufs.at[b], sems[b])

    descs = [start(0), None]
    for k in range(NCHUNK):
        b = k % 2
        if k + 1 < NCHUNK:
            descs[1 - b] = start(k + 1)
        descs[b].wait()
        pltpu.sync_copy(bufs.at[b], out.at[pl.ds(base + k * CHUNK, CHUNK)])


@functools.partial(
    pl.kernel,
    out_type=jax.ShapeDtypeStruct((E_PAD, 16), jnp.float32),
    mesh=_mesh(),
    scratch_types=[
        pltpu.VMEM((NCHUNK, CHUNK), jnp.int32),
        pltpu.VMEM((2, CHUNK, 16), jnp.float32),
        pltpu.SemaphoreType.DMA,
        pltpu.SemaphoreType.DMA,
    ],
    compiler_params=_SC_PARAMS,
    name="sc_gather_16",
)
def _g16(table, src2d, out, idxs, bufs, sem0, sem1):
    wid = lax.axis_index("s") * NCORES + lax.axis_index("c")
    base = wid * QT
    kbase = wid * NCHUNK
    sems = [sem0, sem1]
    pltpu.sync_copy(src2d.at[pl.ds(kbase, NCHUNK)], idxs)

    def start(k):
        b = k % 2
        return pltpu.async_copy(table.at[idxs.at[k]], bufs.at[b], sems[b])

    descs = [start(0), None]
    for k in range(NCHUNK):
        b = k % 2
        if k + 1 < NCHUNK:
            descs[1 - b] = start(k + 1)
        descs[b].wait()
        pltpu.sync_copy(bufs.at[b], out.at[pl.ds(base + k * CHUNK, CHUNK)])


def _make_scatter(nsets, cp):
    out_types = [jax.ShapeDtypeStruct((N_PAD, NCORES * cp), jnp.float32)
                 for _ in range(nsets)]
    scratch = ([pltpu.VMEM_SHARED((N_PAD, cp), jnp.float32)
                for _ in range(nsets)]
               + [pltpu.VMEM((NCHUNK, CHUNK), jnp.int32)
                  for _ in range(nsets)]
               + [pltpu.VMEM((2, CHUNK, cp), jnp.float32),
                  pltpu.SemaphoreType.DMA,
                  pltpu.SemaphoreType.DMA])

    @functools.partial(pl.kernel, out_type=out_types, mesh=_mesh(),
                       scratch_types=scratch, compiler_params=_SC_PARAMS,
                       name=f"sc_scatter_{nsets}x{cp}")
    def sk(*refs):
        ins = refs[:2 * nsets]
        outs = refs[2 * nsets:3 * nsets]
        accs = refs[3 * nsets:4 * nsets]
        idxs = refs[4 * nsets:5 * nsets]
        bufs = refs[5 * nsets]
        sems = [refs[5 * nsets + 1], refs[5 * nsets + 2]]

        cid = lax.axis_index("c")
        sid = lax.axis_index("s")
        wid = sid * NCORES + cid
        base = wid * QT
        kbase = wid * NCHUNK
        zrow = sid * ROWS_T

        for i in range(nsets):
            pltpu.sync_copy(ins[2 * i + 1].at[pl.ds(kbase, NCHUNK)], idxs[i])

        zeros16 = jnp.zeros((16,), jnp.float32)

        def zb(i, carry):
            r = i // (cp // 16)
            j = i % (cp // 16)
            bufs[0, r, pl.ds(j * 16, 16)] = zeros16
            return carry

        lax.fori_loop(0, CHUNK * (cp // 16), zb, 0)
        for a in accs:
            def zs(t, carry, a=a):
                pltpu.sync_copy(bufs.at[0], a.at[pl.ds(zrow + t * CHUNK, CHUNK)])
                return carry
            lax.fori_loop(0, ROWS_T // CHUNK, zs, 0)
        plsc.subcore_barrier()

        chunks = [(i, k) for i in range(nsets) for k in range(NCHUNK)]

        def start(j):
            i, k = chunks[j]
            b = j % 2
            return pltpu.async_copy(
                ins[2 * i].at[pl.ds(base + k * CHUNK, CHUNK), pl.ds(0, cp)],
                bufs.at[b], sems[b])

        descs = [start(0), None]
        for j in range(len(chunks)):
            i, k = chunks[j]
            b = j % 2
            if j + 1 < len(chunks):
                descs[1 - b] = start(j + 1)
            descs[b].wait()
            pltpu.sync_copy(bufs.at[b], accs[i].at[idxs[i].at[k]], add=True)

        plsc.subcore_barrier()
        for i in range(nsets):
            pltpu.sync_copy(accs[i].at[pl.ds(zrow, ROWS_T)],
                            outs[i].at[pl.ds(zrow, ROWS_T),
                                       pl.ds(cid * cp, cp)])

    return sk


BE = 2048
GB = E_PAD // BE // 8


def _unpack16(xjp):
    j = pl.program_id(0) // GB
    lane = lax.broadcasted_iota(jnp.int32, (W, 16), 0)
    col = lax.broadcasted_iota(jnp.int32, (W, 16), 1)
    g = (lane == 16 * j + col).astype(jnp.float32)
    return jnp.dot(xjp, g, preferred_element_type=jnp.float32)


def _make_dense_pre():
    def body(ea_ref, xj_ref, win_ref, bin_ref, wout_ref, out_ref):
        eaT = ea_ref[...]
        xj = _unpack16(xj_ref[...])
        acc = jnp.zeros((BE, 64), jnp.float32)
        for h in range(3):
            s = jnp.maximum(
                lax.dot_general(eaT, win_ref[h], (((0,), (0,)), ((), ())),
                                preferred_element_type=jnp.float32)
                + bin_ref[h], 0.0)
            acc = acc + jnp.dot(s * xj, wout_ref[h],
                                preferred_element_type=jnp.float32)
        out_ref[:, :64] = acc
        out_ref[:, 64:] = jnp.zeros((BE, W - 64), jnp.float32)

    return pl.pallas_call(
        body,
        grid=(E_PAD // BE,),
        in_specs=[
            pl.BlockSpec((8, BE), lambda i: (0, i)),
            pl.BlockSpec((BE, W), lambda i: (i % GB, 0)),
            pl.BlockSpec((3, 8, 16), lambda i: (0, 0, 0)),
            pl.BlockSpec((3, 1, 16), lambda i: (0, 0, 0)),
            pl.BlockSpec((3, 16, 64), lambda i: (0, 0, 0)),
        ],
        out_specs=pl.BlockSpec((BE, W), lambda i: (i, 0)),
        out_shape=jax.ShapeDtypeStruct((E_PAD, W), jnp.float32),
        name="tc_dense_pre",
    )


def _make_dense_end0():
    def body(ea_ref, xjh_ref, xjy_ref, winy_ref, biny_ref, wouty_ref,
             winh_ref, binh_ref, wouth_ref, out_ref):
        eaT = ea_ref[...]
        xjh = xjh_ref[...]
        xjy = _unpack16(xjy_ref[...])
        acc = jnp.zeros((BE, 64), jnp.float32)
        for h in range(3):
            sy = jnp.maximum(
                lax.dot_general(eaT, winy_ref[h], (((0,), (0,)), ((), ())),
                                preferred_element_type=jnp.float32)
                + biny_ref[h], 0.0)
            acc = acc + jnp.dot(sy * xjy, wouty_ref[h],
                                preferred_element_type=jnp.float32)
            sh = jnp.maximum(
                lax.dot_general(eaT, winh_ref[h], (((0,), (0,)), ((), ())),
                                preferred_element_type=jnp.float32)
                + binh_ref[h], 0.0)
            acc = acc + jnp.dot(sh * xjh, wouth_ref[h],
                                preferred_element_type=jnp.float32)
        out_ref[:, :64] = acc
        out_ref[:, 64:] = jnp.zeros((BE, W - 64), jnp.float32)

    return pl.pallas_call(
        body,
        grid=(E_PAD // BE,),
        in_specs=[
            pl.BlockSpec((8, BE), lambda i: (0, i)),
            pl.BlockSpec((BE, W), lambda i: (i, 0)),
            pl.BlockSpec((BE, W), lambda i: (i % GB, 0)),
            pl.BlockSpec((3, 8, 16), lambda i: (0, 0, 0)),
            pl.BlockSpec((3, 1, 16), lambda i: (0, 0, 0)),
            pl.BlockSpec((3, 16, 64), lambda i: (0, 0, 0)),
            pl.BlockSpec((3, 8, W), lambda i: (0, 0, 0)),
            pl.BlockSpec((3, 1, W), lambda i: (0, 0, 0)),
            pl.BlockSpec((3, W, 64), lambda i: (0, 0, 0)),
        ],
        out_specs=pl.BlockSpec((BE, W), lambda i: (i, 0)),
        out_shape=jax.ShapeDtypeStruct((E_PAD, W), jnp.float32),
        name="tc_dense_end0",
    )


def _make_dense_end1():
    def body(ea_ref, xj_ref, win_ref, bin_ref, wout_ref, out_ref):
        eaT = ea_ref[...]
        xj = xj_ref[...]
        acc = jnp.zeros((BE, 16), jnp.float32)
        for h in range(3):
            s = jnp.maximum(
                lax.dot_general(eaT, win_ref[h], (((0,), (0,)), ((), ())),
                                preferred_element_type=jnp.float32)
                + bin_ref[h], 0.0)
            acc = acc + jnp.dot(s * xj, wout_ref[h],
                                preferred_element_type=jnp.float32)
        out_ref[:, :16] = acc
        out_ref[:, 16:] = jnp.zeros((BE, W - 16), jnp.float32)

    return pl.pallas_call(
        body,
        grid=(E_PAD // BE,),
        in_specs=[
            pl.BlockSpec((8, BE), lambda i: (0, i)),
            pl.BlockSpec((BE, W), lambda i: (i, 0)),
            pl.BlockSpec((3, 8, W), lambda i: (0, 0, 0)),
            pl.BlockSpec((3, 1, W), lambda i: (0, 0, 0)),
            pl.BlockSpec((3, W, 16), lambda i: (0, 0, 0)),
        ],
        out_specs=pl.BlockSpec((BE, W), lambda i: (i, 0)),
        out_shape=jax.ShapeDtypeStruct((E_PAD, W), jnp.float32),
        name="tc_dense_end1",
    )


BN = 512


def _knn_body(fp_ref, cpos_ref, cy_ref, out_ref):
    f0 = fp_ref[:, 0:1]
    f1 = fp_ref[:, 1:2]
    c0 = cpos_ref[0:1, :]
    c1 = cpos_ref[1:2, :]
    d2 = (f0 - c0) ** 2 + (f1 - c1) ** 2
    big = jnp.float32(jnp.inf)
    m1 = jnp.min(d2, axis=1, keepdims=True)
    d2a = jnp.where(d2 > m1, d2, big)
    m2 = jnp.min(d2a, axis=1, keepdims=True)
    d2b = jnp.where(d2a > m2, d2a, big)
    m3 = jnp.min(d2b, axis=1, keepdims=True)
    sel = (d2 <= m3).astype(jnp.float32)
    w = sel / jnp.maximum(d2, 1e-16)
    num = jnp.dot(w, cy_ref[...], preferred_element_type=jnp.float32)
    den = jnp.sum(w, axis=1, keepdims=True)
    out_ref[...] = num / den


_knn = pl.pallas_call(
    _knn_body,
    grid=(N_PAD // BN,),
    in_specs=[
        pl.BlockSpec((BN, 2), lambda i: (i, 0)),
        pl.BlockSpec((8, NC_PAD), lambda i: (0, 0)),
        pl.BlockSpec((NC_PAD, 8), lambda i: (0, 0)),
    ],
    out_specs=pl.BlockSpec((BN, 8), lambda i: (i, 0)),
    out_shape=jax.ShapeDtypeStruct((N_PAD, 8), jnp.float32),
    name="tc_knn",
)


def _finalize2_body(p1_ref, b1_ref, p2_ref, b2_ref, out_ref):
    a1 = jnp.tanh(p1_ref[:, :64] + p1_ref[:, 64:] + b1_ref[...])
    a2 = jnp.tanh(p2_ref[:, :64] + p2_ref[:, 64:] + b2_ref[...])
    out_ref[:, :64] = jnp.maximum(a1, 0.0)
    out_ref[:, 64:] = jnp.maximum(a2, 0.0)


_finalize2 = pl.pallas_call(
    _finalize2_body,
    grid=(N_PAD // BN,),
    in_specs=[
        pl.BlockSpec((BN, W), lambda i: (i, 0)),
        pl.BlockSpec((1, 64), lambda i: (0, 0)),
        pl.BlockSpec((BN, W), lambda i: (i, 0)),
        pl.BlockSpec((1, 64), lambda i: (0, 0)),
    ],
    out_specs=pl.BlockSpec((BN, W), lambda i: (i, 0)),
    out_shape=jax.ShapeDtypeStruct((N_PAD, W), jnp.float32),
    name="tc_finalize2",
)


def _finalize1_body(p_ref, b_ref, out_ref):
    out_ref[...] = jnp.tanh(p_ref[:, :16] + p_ref[:, 16:] + b_ref[...])


_finalize1 = pl.pallas_call(
    _finalize1_body,
    grid=(N_PAD // BN,),
    in_specs=[
        pl.BlockSpec((BN, 32), lambda i: (i, 0)),
        pl.BlockSpec((1, 16), lambda i: (0, 0)),
    ],
    out_specs=pl.BlockSpec((BN, 16), lambda i: (i, 0)),
    out_shape=jax.ShapeDtypeStruct((N_PAD, 16), jnp.float32),
    name="tc_finalize1",
)


def _pad2(a, rows, cols):
    return jnp.pad(a, ((0, rows - a.shape[0]), (0, cols - a.shape[1])))


def _split_w(p, cin):
    win = jnp.stack([p['Win'][:, h::3] for h in range(3)])
    bin_ = jnp.stack([p['bin'][h::3][None, :] for h in range(3)])
    wout = jnp.stack([p['Wout'][h::3, :] for h in range(3)])
    assert win.shape[2] == cin
    return win, bin_, wout


def _prep_pre(p):
    win, bin_, wout = _split_w(p, 6)
    return (jnp.pad(win, ((0, 0), (0, 2), (0, 10))),
            jnp.pad(bin_, ((0, 0), (0, 0), (0, 10))),
            jnp.pad(wout, ((0, 0), (0, 10), (0, 0))),
            p['bout'][None, :])


def _prep_end0(p):
    win, bin_, wout = _split_w(p, 131)
    winy = jnp.pad(win[:, :, :3], ((0, 0), (0, 2), (0, 13)))
    biny = jnp.pad(bin_[:, :, :3], ((0, 0), (0, 0), (0, 13)))
    wouty = jnp.pad(wout[:, :3, :], ((0, 0), (0, 13), (0, 0)))
    winh = jnp.pad(win[:, :, 3:], ((0, 0), (0, 2), (0, 0)))
    binh = bin_[:, :, 3:]
    wouth = wout[:, 3:, :]
    return winy, biny, wouty, winh, binh, wouth, p['bout'][None, :]


def _prep_end1(p):
    win, bin_, wout = _split_w(p, 128)
    return (jnp.pad(win, ((0, 0), (0, 2), (0, 0))),
            bin_,
            jnp.pad(wout, ((0, 0), (0, 0), (0, 16 - wout.shape[2]))),
            jnp.pad(p['bout'][None, :], ((0, 0), (0, 13))))


def kernel(x, sdf, edge_index, edge_indexA2, edge_attr, edge_attrA2,
           coarse_nodes, coarse_y, params):
    i32 = jnp.int32
    pe = E_PAD - N_EDGES

    pad_src = (jnp.arange(pe, dtype=i32) * 97) % N_NODES
    pad_dst = N_NODES + jnp.arange(pe, dtype=i32) % (N_PAD - N_NODES)

    def prep_edges(ei, ea):
        srcf = jnp.concatenate([ei[0].astype(i32), pad_src])
        src = srcf.reshape(E_PAD // CHUNK, CHUNK)
        srcp = srcf.reshape(8, E_PAD // 8).T.reshape(E_PAD // CHUNK, CHUNK)
        dst = jnp.concatenate([ei[1].astype(i32), pad_dst]
                              ).reshape(E_PAD // CHUNK, CHUNK)
        eaT = _pad2(ea.T.astype(jnp.float32), 8, E_PAD)
        return src, srcp, dst, eaT

    src1, src1p, dst1, ea1 = prep_edges(edge_index, edge_attr)
    src2, src2p, dst2, ea2 = prep_edges(edge_indexA2, edge_attrA2)

    sc2_64 = _make_scatter(2, 64)
    sc1_16 = _make_scatter(1, 16)
    d_pre = _make_dense_pre()
    d_end0 = _make_dense_end0()
    d_end1 = _make_dense_end1()

    wp1 = _prep_pre(params['pre0'][0])
    wp2 = _prep_pre(params['pre0'][1])
    we1 = _prep_end0(params['end0'][0])
    we2 = _prep_end0(params['end0'][1])
    wl = _prep_end1(params['end1'][0])

    t0 = _pad2(jnp.concatenate([x, sdf], axis=1), N_PAD, 16)
    xj1 = _g16(t0, src1p).reshape(E_PAD // 8, W)
    xj2 = _g16(t0, src2p).reshape(E_PAD // 8, W)
    c1 = d_pre(ea1, xj1, wp1[0], wp1[1], wp1[2])
    c2 = d_pre(ea2, xj2, wp2[0], wp2[1], wp2[2])
    p1, p2 = sc2_64(c1, dst1, c2, dst2)
    h0 = _finalize2(p1, wp1[3], p2, wp2[3])

    fpos = _pad2(x[:, :2], N_PAD, 2)
    cpos = _pad2(coarse_nodes.T, 8, NC_PAD)
    cpos = cpos + jnp.pad(jnp.full((8, NC_PAD - N_COARSE), 1e30),
                          ((0, 0), (N_COARSE, 0)))
    cy = _pad2(coarse_y, NC_PAD, 8)
    yk = _knn(fpos, cpos, cy)
    yk16 = _pad2(yk, N_PAD, 16)

    xh1 = _g128(h0, src1)
    xy1 = _g16(yk16, src1p).reshape(E_PAD // 8, W)
    c1 = d_end0(ea1, xh1, xy1, *we1[:6])
    xh2 = _g128(h0, src2)
    xy2 = _g16(yk16, src2p).reshape(E_PAD // 8, W)
    c2 = d_end0(ea2, xh2, xy2, *we2[:6])
    p1, p2 = sc2_64(c1, dst1, c2, dst2)
    h1 = _finalize2(p1, we1[6], p2, we2[6])

    xj = _g128(h1, src1)
    c = d_end1(ea1, xj, wl[0], wl[1], wl[2])
    (p,) = sc1_16(c, dst1)
    o = _finalize1(p, wl[3])
    return o[:N_NODES, :3]
